# Initial kernel scaffold; baseline (speedup 1.0000x reference)
import jax
import jax.numpy as jnp
from jax.experimental import pallas as pl


def kernel(pose_feats, edge_index, edge_attr, node_timestamps, batch, edge_encoder, node_encoder, edge_classifier, edge_update, create_past_msgs, create_future_msgs, combine_future_past):
    raise NotImplementedError("write your pallas kernel here")


# speedup vs baseline: 2.1570x; 2.1570x over previous
import functools

import jax
import jax.numpy as jnp
from jax import lax
from jax.experimental import pallas as pl
from jax.experimental.pallas import tpu as pltpu
from jax.experimental.pallas import tpu_sc as plsc

_NC = 2
_NS = 16
_NW = _NC * _NS
_CH = 128

_F32 = jnp.float32


def _prep(mlp):
    return [(w.T, b[None, :]) for (w, b) in mlp]


def _mlp_tc(x, layers, block_rows, table128=False):
    m, din = x.shape
    n_layers = len(layers)
    dh_out = layers[-1][0].shape[1]
    dout = 128 if table128 else dh_out

    def body(x_ref, *refs):
        out_ref = refs[-1]
        h = x_ref[...]
        for i in range(n_layers):
            w = refs[2 * i][...]
            b = refs[2 * i + 1][...]
            h = jnp.dot(h, w, preferred_element_type=_F32) + b
            if i < n_layers - 1:
                h = jnp.maximum(h, 0.0)
        if table128:
            h = jnp.concatenate(
                [h, h, jnp.zeros((h.shape[0], 128 - 2 * dh_out), _F32)],
                axis=1)
        out_ref[...] = h

    in_specs = [pl.BlockSpec((block_rows, din), lambda i: (i, 0))]
    args = [x]
    for (w, b) in layers:
        in_specs.append(pl.BlockSpec(w.shape, lambda i: (0, 0)))
        in_specs.append(pl.BlockSpec(b.shape, lambda i: (0, 0)))
        args.extend([w, b])
    return pl.pallas_call(
        body,
        grid=(m // block_rows,),
        in_specs=in_specs,
        out_specs=pl.BlockSpec((block_rows, dout), lambda i: (i, 0)),
        out_shape=jax.ShapeDtypeStruct((m, dout), _F32),
        compiler_params=pltpu.CompilerParams(
            dimension_semantics=("arbitrary",)),
    )(*args)


def _edge_stage(pack_i, pack_j, ea, eu, fu, pa, dx, block_rows, e_pad,
                last=False):
    e = ea.shape[0]
    dea = ea.shape[1]
    dt = pack_i.shape[1]
    (euw0, eub0), (euw1, eub1), (euw2, eub2) = eu
    (fuw0, fub0), (fuw1, fub1) = fu
    (paw0, pab0), (paw1, pab1) = pa
    dup = euw2.shape[1]
    dmsg = fuw1.shape[1]
    dh = dmsg // 2
    dmid = euw0.shape[1]

    z = jnp.zeros((dt - 2 * dx, dmid), _F32)
    a1 = jnp.concatenate([euw0[:dx], jnp.zeros((dx, dmid), _F32), z])
    a2 = jnp.concatenate([euw0[dx:2 * dx], jnp.zeros((dx, dmid), _F32), z])
    a3 = euw0[2 * dx:]
    f1 = jnp.concatenate([fuw0[:dx], fuw0[dx + dup:], z])
    f2 = fuw0[dx:dx + dup]
    p1 = jnp.concatenate([paw0[:dx], paw0[dx + dup:], z])
    p2 = paw0[dx:dx + dup]

    def body(*refs):
        if last:
            (pi_r, pj_r, ea_r,
             a1_r, a2_r, a3_r, ab0, a4, ab1, a5, ab2, upd_r) = refs
        else:
            (pi_r, pj_r, ea_r,
             a1_r, a2_r, a3_r, ab0, a4, ab1, a5, ab2,
             f1_r, f2_r, fb0, f4, fb1,
             p1_r, p2_r, pb0, p4, pb1,
             upd_r, pa_r, fu_r) = refs
        pi = pi_r[...]
        pj = pj_r[...]
        dot = lambda a, b: jnp.dot(a, b[...], preferred_element_type=_F32)
        h = jnp.maximum(dot(pi, a1_r) + dot(pj, a2_r) + dot(ea_r[...], a3_r)
                        + ab0[...], 0.0)
        h = jnp.maximum(dot(h, a4) + ab1[...], 0.0)
        upd = dot(h, a5) + ab2[...]
        upd_r[...] = upd
        if last:
            return
        f = jnp.maximum(dot(pi, f1_r) + dot(upd, f2_r) + fb0[...], 0.0)
        fut = dot(f, f4) + fb1[...]
        fu_r[0] = fut[:, :dh]
        fu_r[1] = fut[:, dh:]
        p = jnp.maximum(dot(pj, p1_r) + dot(upd, p2_r) + pb0[...], 0.0)
        pst = dot(p, p4) + pb1[...]
        pa_r[0] = pst[:, :dh]
        pa_r[1] = pst[:, dh:]

    row_spec = lambda d: pl.BlockSpec((block_rows, d), lambda i: (i, 0))
    full_spec = lambda a: pl.BlockSpec(a.shape, lambda i: (0, 0))
    eu_w = [a1, a2, a3, eub0, euw1, eub1, euw2, eub2]
    if last:
        args = [pack_i, pack_j, ea] + eu_w
        in_specs = ([row_spec(dt)] * 2 + [row_spec(dea)]
                    + [full_spec(a) for a in eu_w])
        out_sds = jax.ShapeDtypeStruct((e, dup), _F32)
        out_specs = row_spec(dup)
    else:
        wargs = eu_w + [f1, f2, fub0, fuw1, fub1,
                        p1, p2, pab0, paw1, pab1]
        args = [pack_i, pack_j, ea] + wargs
        in_specs = ([row_spec(dt)] * 2 + [row_spec(dea)]
                    + [full_spec(a) for a in wargs])
        stk_spec = pl.BlockSpec((2, block_rows, dh), lambda i: (0, i, 0))
        out_sds = [jax.ShapeDtypeStruct((e, dup), _F32)] + \
                  [jax.ShapeDtypeStruct((2, e_pad, dh), _F32)] * 2
        out_specs = [row_spec(dup)] + [stk_spec] * 2
    return pl.pallas_call(
        body,
        grid=(e // block_rows,),
        in_specs=in_specs,
        out_specs=out_specs,
        out_shape=out_sds,
        compiler_params=pltpu.CompilerParams(
            dimension_semantics=("arbitrary",)),
    )(*args)


def _combine_tc(mp, mf, tab0, layers, block_rows, n_rows):
    _, n, dh = mp.shape
    (w0, b0), (w1, b1), (w2, b2) = layers
    s0, s1, s2, s3 = w0[:dh], w0[dh:2 * dh], w0[2 * dh:3 * dh], w0[3 * dh:]
    dout = w2.shape[1]

    def body(a_r, b_r, c_r, d_r, ix_r, t0, t1, t2, t3, tb0, t4, tb1, t5,
             tb2, out_r):
        dot = lambda a, b: jnp.dot(a, b[...], preferred_element_type=_F32)
        h = jnp.maximum(dot(a_r[0], t0) + dot(b_r[0], t1)
                        + dot(c_r[0], t2) + dot(d_r[0], t3)
                        + tb0[...], 0.0)
        h = jnp.maximum(dot(h, t4) + tb1[...], 0.0)
        h = dot(h, t5) + tb2[...]
        ix = ix_r[...][:, :dout]
        out_r[...] = jnp.concatenate(
            [h, ix, jnp.zeros((h.shape[0], 128 - 2 * dout), _F32)], axis=1)

    lo_spec = pl.BlockSpec((1, block_rows, dh), lambda i: (0, i, 0))
    hi_spec = pl.BlockSpec((1, block_rows, dh), lambda i: (1, i, 0))
    row_spec = lambda d: pl.BlockSpec((block_rows, d), lambda i: (i, 0))
    full_spec = lambda a: pl.BlockSpec(a.shape, lambda i: (0, 0))
    wargs = [s0, s1, s2, s3, b0, w1, b1, w2, b2]
    return pl.pallas_call(
        body,
        grid=(n_rows // block_rows,),
        in_specs=[lo_spec, hi_spec, lo_spec, hi_spec, row_spec(128)]
                 + [full_spec(a) for a in wargs],
        out_specs=row_spec(128),
        out_shape=jax.ShapeDtypeStruct((n, 128), _F32),
        compiler_params=pltpu.CompilerParams(
            dimension_semantics=("arbitrary",)),
    )(mp, mp, mf, mf, tab0, *wargs)


def _sc_gather2(table, idx_a, idx_b):
    n, dt = table.shape
    e = idx_a.shape[0]
    epw = e // _NW
    nfull = epw // _CH
    tail = epw - nfull * _CH
    mesh = plsc.VectorSubcoreMesh(core_axis_name="c", subcore_axis_name="s",
                                  num_cores=_NC, num_subcores=_NS)

    @functools.partial(
        pl.kernel,
        out_type=(jax.ShapeDtypeStruct((e, dt), _F32),
                  jax.ShapeDtypeStruct((e, dt), _F32)),
        mesh=mesh,
        scratch_types=[
            pltpu.VMEM((epw,), jnp.int32),
            pltpu.VMEM((epw,), jnp.int32),
            pltpu.VMEM((_CH, dt), _F32),
            pltpu.VMEM((_CH, dt), _F32),
            pltpu.SemaphoreType.DMA,
            pltpu.SemaphoreType.DMA,
        ],
    )
    def k(table_h, ia_h, ib_h, oa_h, ob_h, ia_v, ib_v, buf_a, buf_b,
          sem_a, sem_b):
        w = lax.axis_index("s") * _NC + lax.axis_index("c")
        e0 = w * epw
        pltpu.sync_copy(ia_h.at[pl.ds(e0, epw)], ia_v)
        pltpu.sync_copy(ib_h.at[pl.ds(e0, epw)], ib_v)

        def chunk(off, sz):
            cp_a = pltpu.async_copy(
                table_h.at[ia_v.at[pl.ds(off, sz)]],
                buf_a.at[pl.ds(0, sz)], sem_a)
            cp_b = pltpu.async_copy(
                table_h.at[ib_v.at[pl.ds(off, sz)]],
                buf_b.at[pl.ds(0, sz)], sem_b)
            cp_a.wait()
            pltpu.sync_copy(buf_a.at[pl.ds(0, sz)],
                            oa_h.at[pl.ds(e0 + off, sz)])
            cp_b.wait()
            pltpu.sync_copy(buf_b.at[pl.ds(0, sz)],
                            ob_h.at[pl.ds(e0 + off, sz)])

        def body(g, carry):
            chunk(g * _CH, _CH)
            return carry

        lax.fori_loop(0, nfull, body, 0)
        if tail:
            chunk(nfull * _CH, tail)

    return k(table, idx_a, idx_b)


def _sc_scatter(msg, idx, n_pad):
    ep2, dh = msg.shape
    ep = ep2 // 2
    r, ch = idx.shape
    rpt = r // _NS
    nrpt = n_pad // _NS
    ib = 8
    assert rpt % ib == 0
    mesh = plsc.VectorSubcoreMesh(core_axis_name="c", subcore_axis_name="s",
                                  num_cores=_NC, num_subcores=_NS)

    nz = nrpt // ch
    zt = nrpt - nz * ch
    zeros = jnp.zeros((ch, dh), _F32)

    @functools.partial(
        pl.kernel,
        out_type=jax.ShapeDtypeStruct((2 * n_pad, dh), _F32),
        mesh=mesh,
        scratch_types=[
            pltpu.VMEM_SHARED((n_pad + 8, dh), _F32),
            pltpu.VMEM((ib, ch), jnp.int32),
            pltpu.VMEM((ch, dh), _F32),
        ],
        compiler_params=pltpu.CompilerParams(use_tc_tiling_on_sc=False),
    )
    def k(m_h, idx_h, z_h, o_h, acc_s, idx_v, mbuf):
        c = lax.axis_index("c")
        s = lax.axis_index("s")
        m0 = c * ep
        o0 = c * n_pad

        pltpu.sync_copy(z_h, mbuf)

        def zacc(i, carry):
            pltpu.sync_copy(mbuf, acc_s.at[pl.ds(s * nrpt + i * ch, ch)])
            return carry

        lax.fori_loop(0, nz, zacc, 0)
        if zt:
            pltpu.sync_copy(mbuf.at[pl.ds(0, zt)],
                            acc_s.at[pl.ds(s * nrpt + nz * ch, zt)])
        plsc.subcore_barrier()

        def body(o, carry):
            pltpu.sync_copy(idx_h.at[pl.ds(s * rpt + o * ib, ib)], idx_v)
            for j in range(ib):
                g = o * ib + j
                pltpu.sync_copy(
                    m_h.at[pl.ds(m0 + (s * rpt + g) * ch, ch)], mbuf)
                pltpu.sync_copy(mbuf, acc_s.at[idx_v.at[j]], add=True)
            return carry

        lax.fori_loop(0, rpt // ib, body, 0)
        plsc.subcore_barrier()

        def ebody(i, carry):
            pltpu.sync_copy(acc_s.at[pl.ds(s * nrpt + i * ch, ch)], mbuf)
            pltpu.sync_copy(mbuf,
                            o_h.at[pl.ds(o0 + s * nrpt + i * ch, ch)])
            return carry

        lax.fori_loop(0, nz, ebody, 0)
        if zt:
            pltpu.sync_copy(acc_s.at[pl.ds(s * nrpt + nz * ch, zt)],
                            mbuf.at[pl.ds(0, zt)])
            pltpu.sync_copy(mbuf.at[pl.ds(0, zt)],
                            o_h.at[pl.ds(o0 + s * nrpt + nz * ch, zt)])

    return k(msg, idx, zeros)


_DEPTH = 6


def kernel(pose_feats, edge_index, edge_attr, node_timestamps, batch,
           edge_encoder, node_encoder, edge_classifier, edge_update,
           create_past_msgs, create_future_msgs, combine_future_past):
    n = pose_feats.shape[0]
    e = edge_index.shape[1]
    rows = edge_index[0].astype(jnp.int32)
    cols = edge_index[1].astype(jnp.int32)

    e_pad = -(-e // (_NS * _CH * 8)) * (_NS * _CH * 8)
    n_pad = -(-n // (_NS * 8)) * (_NS * 8)
    pad = jnp.full((e_pad - e,), n_pad, jnp.int32)
    cols_p = jnp.concatenate([cols, pad]).reshape(e_pad // _CH, _CH)
    rows_p = jnp.concatenate([rows, pad]).reshape(e_pad // _CH, _CH)

    node_l = _prep(node_encoder)
    edge_l = _prep(edge_encoder)
    cls_l = _prep(edge_classifier)
    eu_l = _prep(edge_update)
    fu_l = _prep(create_future_msgs)
    pa_l = _prep(create_past_msgs)
    cb_l = _prep(combine_future_past)

    nb = 2000
    ebk = 4000
    dx = node_l[-1][0].shape[1]

    tab0 = _mlp_tc(pose_feats, node_l, nb, table128=True)
    ea = _mlp_tc(edge_attr, edge_l, ebk)

    pack_i, pack_j = _sc_gather2(tab0, cols, rows)
    for depth in range(_DEPTH):
        if depth == _DEPTH - 1:
            ea = _edge_stage(pack_i, pack_j, ea, eu_l, fu_l, pa_l, dx,
                             ebk, e_pad, last=True)
            break
        upd, pa_stk, fu_stk = _edge_stage(
            pack_i, pack_j, ea, eu_l, fu_l, pa_l, dx, ebk, e_pad)
        dh = pa_stk.shape[2]
        mp = _sc_scatter(pa_stk.reshape(-1, dh), cols_p, n_pad)
        mf = _sc_scatter(fu_stk.reshape(-1, dh), rows_p, n_pad)
        ea = upd
        x_tab = _combine_tc(mp.reshape(2, n_pad, dh),
                            mf.reshape(2, n_pad, dh), tab0, cb_l, nb, n)
        pack_i, pack_j = _sc_gather2(x_tab, cols, rows)

    logits = _mlp_tc(ea, cls_l, ebk)
    x_enc = lax.slice(tab0, (0, 0), (n, dx))
    return (logits, x_enc)


# speedup vs baseline: 2.2079x; 1.0236x over previous
import functools

import jax
import jax.numpy as jnp
from jax import lax
from jax.experimental import pallas as pl
from jax.experimental.pallas import tpu as pltpu
from jax.experimental.pallas import tpu_sc as plsc

_NC = 2
_NS = 16
_NW = _NC * _NS
_CH = 128

_F32 = jnp.float32


def _prep(mlp):
    return [(w.T, b[None, :]) for (w, b) in mlp]


def _mlp_tc(x, layers, block_rows, table128=False):
    m, din = x.shape
    n_layers = len(layers)
    dh_out = layers[-1][0].shape[1]
    dout = 128 if table128 else dh_out

    def body(x_ref, *refs):
        out_ref = refs[-1]
        h = x_ref[...]
        for i in range(n_layers):
            w = refs[2 * i][...]
            b = refs[2 * i + 1][...]
            h = jnp.dot(h, w, preferred_element_type=_F32) + b
            if i < n_layers - 1:
                h = jnp.maximum(h, 0.0)
        if table128:
            h = jnp.concatenate(
                [h, h, jnp.zeros((h.shape[0], 128 - 2 * dh_out), _F32)],
                axis=1)
        out_ref[...] = h

    in_specs = [pl.BlockSpec((block_rows, din), lambda i: (i, 0))]
    args = [x]
    for (w, b) in layers:
        in_specs.append(pl.BlockSpec(w.shape, lambda i: (0, 0)))
        in_specs.append(pl.BlockSpec(b.shape, lambda i: (0, 0)))
        args.extend([w, b])
    return pl.pallas_call(
        body,
        grid=(m // block_rows,),
        in_specs=in_specs,
        out_specs=pl.BlockSpec((block_rows, dout), lambda i: (i, 0)),
        out_shape=jax.ShapeDtypeStruct((m, dout), _F32),
        compiler_params=pltpu.CompilerParams(
            dimension_semantics=("arbitrary",)),
    )(*args)


def _edge_stage(pack_i, pack_j, ea, eu, fu, pa, dx, block_rows, e_pad,
                last=False):
    e = ea.shape[0]
    dea = ea.shape[1]
    dt = pack_i.shape[1]
    (euw0, eub0), (euw1, eub1), (euw2, eub2) = eu
    (fuw0, fub0), (fuw1, fub1) = fu
    (paw0, pab0), (paw1, pab1) = pa
    dup = euw2.shape[1]
    dmsg = fuw1.shape[1]
    dh = dmsg // 2
    dmid = euw0.shape[1]

    z = jnp.zeros((dt - 2 * dx, dmid), _F32)
    a1 = jnp.concatenate([euw0[:dx], jnp.zeros((dx, dmid), _F32), z])
    a2 = jnp.concatenate([euw0[dx:2 * dx], jnp.zeros((dx, dmid), _F32), z])
    a3 = euw0[2 * dx:]
    f1 = jnp.concatenate([fuw0[:dx], fuw0[dx + dup:], z])
    f2 = fuw0[dx:dx + dup]
    p1 = jnp.concatenate([paw0[:dx], paw0[dx + dup:], z])
    p2 = paw0[dx:dx + dup]

    def body(*refs):
        if last:
            (pi_r, pj_r, ea_r,
             a1_r, a2_r, a3_r, ab0, a4, ab1, a5, ab2, upd_r) = refs
        else:
            (pi_r, pj_r, ea_r,
             a1_r, a2_r, a3_r, ab0, a4, ab1, a5, ab2,
             f1_r, f2_r, fb0, f4, fb1,
             p1_r, p2_r, pb0, p4, pb1,
             upd_r, pa_r, fu_r) = refs
        pi = pi_r[...]
        pj = pj_r[...]
        dot = lambda a, b: jnp.dot(a, b[...], preferred_element_type=_F32)
        h = jnp.maximum(dot(pi, a1_r) + dot(pj, a2_r) + dot(ea_r[...], a3_r)
                        + ab0[...], 0.0)
        h = jnp.maximum(dot(h, a4) + ab1[...], 0.0)
        upd = dot(h, a5) + ab2[...]
        upd_r[...] = upd
        if last:
            return
        f = jnp.maximum(dot(pi, f1_r) + dot(upd, f2_r) + fb0[...], 0.0)
        fut = dot(f, f4) + fb1[...]
        fu_r[0] = fut[:, :dh]
        fu_r[1] = fut[:, dh:]
        p = jnp.maximum(dot(pj, p1_r) + dot(upd, p2_r) + pb0[...], 0.0)
        pst = dot(p, p4) + pb1[...]
        pa_r[0] = pst[:, :dh]
        pa_r[1] = pst[:, dh:]

    row_spec = lambda d: pl.BlockSpec((block_rows, d), lambda i: (i, 0))
    full_spec = lambda a: pl.BlockSpec(a.shape, lambda i: (0, 0))
    eu_w = [a1, a2, a3, eub0, euw1, eub1, euw2, eub2]
    if last:
        args = [pack_i, pack_j, ea] + eu_w
        in_specs = ([row_spec(dt)] * 2 + [row_spec(dea)]
                    + [full_spec(a) for a in eu_w])
        out_sds = jax.ShapeDtypeStruct((e, dup), _F32)
        out_specs = row_spec(dup)
    else:
        wargs = eu_w + [f1, f2, fub0, fuw1, fub1,
                        p1, p2, pab0, paw1, pab1]
        args = [pack_i, pack_j, ea] + wargs
        in_specs = ([row_spec(dt)] * 2 + [row_spec(dea)]
                    + [full_spec(a) for a in wargs])
        stk_spec = pl.BlockSpec((2, block_rows, dh), lambda i: (0, i, 0))
        out_sds = [jax.ShapeDtypeStruct((e, dup), _F32)] + \
                  [jax.ShapeDtypeStruct((2, e_pad, dh), _F32)] * 2
        out_specs = [row_spec(dup)] + [stk_spec] * 2
    return pl.pallas_call(
        body,
        grid=(e // block_rows,),
        in_specs=in_specs,
        out_specs=out_specs,
        out_shape=out_sds,
        compiler_params=pltpu.CompilerParams(
            dimension_semantics=("arbitrary",)),
    )(*args)


def _combine_tc(mp, mf, tab0, layers, block_rows, n_rows):
    _, n, dh = mp.shape
    (w0, b0), (w1, b1), (w2, b2) = layers
    s0, s1, s2, s3 = w0[:dh], w0[dh:2 * dh], w0[2 * dh:3 * dh], w0[3 * dh:]
    dout = w2.shape[1]

    def body(a_r, b_r, c_r, d_r, ix_r, t0, t1, t2, t3, tb0, t4, tb1, t5,
             tb2, out_r):
        dot = lambda a, b: jnp.dot(a, b[...], preferred_element_type=_F32)
        h = jnp.maximum(dot(a_r[0], t0) + dot(b_r[0], t1)
                        + dot(c_r[0], t2) + dot(d_r[0], t3)
                        + tb0[...], 0.0)
        h = jnp.maximum(dot(h, t4) + tb1[...], 0.0)
        h = dot(h, t5) + tb2[...]
        ix = ix_r[...][:, :dout]
        out_r[...] = jnp.concatenate(
            [h, ix, jnp.zeros((h.shape[0], 128 - 2 * dout), _F32)], axis=1)

    lo_spec = pl.BlockSpec((1, block_rows, dh), lambda i: (0, i, 0))
    hi_spec = pl.BlockSpec((1, block_rows, dh), lambda i: (1, i, 0))
    row_spec = lambda d: pl.BlockSpec((block_rows, d), lambda i: (i, 0))
    full_spec = lambda a: pl.BlockSpec(a.shape, lambda i: (0, 0))
    wargs = [s0, s1, s2, s3, b0, w1, b1, w2, b2]
    return pl.pallas_call(
        body,
        grid=(n_rows // block_rows,),
        in_specs=[lo_spec, hi_spec, lo_spec, hi_spec, row_spec(128)]
                 + [full_spec(a) for a in wargs],
        out_specs=row_spec(128),
        out_shape=jax.ShapeDtypeStruct((n, 128), _F32),
        compiler_params=pltpu.CompilerParams(
            dimension_semantics=("arbitrary",)),
    )(mp, mp, mf, mf, tab0, *wargs)


def _sc_gather2(table, idx_a, idx_b):
    n, dt = table.shape
    e = idx_a.shape[0]
    epw = e // _NW
    nch = -(-epw // _CH)
    nch += nch % 2
    mesh = plsc.VectorSubcoreMesh(core_axis_name="c", subcore_axis_name="s",
                                  num_cores=_NC, num_subcores=_NS)

    @functools.partial(
        pl.kernel,
        out_type=(jax.ShapeDtypeStruct((e, dt), _F32),
                  jax.ShapeDtypeStruct((e, dt), _F32)),
        mesh=mesh,
        scratch_types=[
            pltpu.VMEM((epw,), jnp.int32),
            pltpu.VMEM((epw,), jnp.int32),
            pltpu.VMEM((_CH, dt), _F32),
            pltpu.VMEM((_CH, dt), _F32),
            pltpu.SemaphoreType.DMA,
            pltpu.SemaphoreType.DMA,
        ],
    )
    def k(table_h, ia_h, ib_h, oa_h, ob_h, ia_v, ib_v, buf_a, buf_b,
          sem_a, sem_b):
        w = lax.axis_index("s") * _NC + lax.axis_index("c")
        e0 = w * epw
        pltpu.sync_copy(ia_h.at[pl.ds(e0, epw)], ia_v)
        pltpu.sync_copy(ib_h.at[pl.ds(e0, epw)], ib_v)

        def off(g):
            return jnp.minimum(g * _CH, epw - _CH)

        def run(idx_v, o_h):
            def fire(g, buf, sem):
                return pltpu.async_copy(
                    table_h.at[idx_v.at[pl.ds(off(g), _CH)]], buf, sem)

            def drain(buf, sem):
                pltpu.make_async_copy(table_h.at[pl.ds(0, _CH)], buf,
                                      sem).wait()

            def write(g, buf):
                pltpu.sync_copy(buf, o_h.at[pl.ds(e0 + off(g), _CH)])

            fire(0, buf_a, sem_a)
            fire(1, buf_b, sem_b)

            def body(o, carry):
                g0 = 2 * o
                drain(buf_a, sem_a)
                write(g0, buf_a)
                fire(g0 + 2, buf_a, sem_a)
                drain(buf_b, sem_b)
                write(g0 + 1, buf_b)
                fire(g0 + 3, buf_b, sem_b)
                return carry

            lax.fori_loop(0, nch // 2 - 1, body, 0)
            drain(buf_a, sem_a)
            write(nch - 2, buf_a)
            drain(buf_b, sem_b)
            write(nch - 1, buf_b)

        run(ia_v, oa_h)
        run(ib_v, ob_h)

    return k(table, idx_a, idx_b)


def _sc_scatter(msg, idx, n_pad):
    ep2, dh = msg.shape
    ep = ep2 // 2
    r, ch = idx.shape
    rpt = r // _NS
    nrpt = n_pad // _NS
    ib = 8
    assert rpt % ib == 0
    mesh = plsc.VectorSubcoreMesh(core_axis_name="c", subcore_axis_name="s",
                                  num_cores=_NC, num_subcores=_NS)

    nz = nrpt // ch
    zt = nrpt - nz * ch
    zeros = jnp.zeros((ch, dh), _F32)

    @functools.partial(
        pl.kernel,
        out_type=jax.ShapeDtypeStruct((2 * n_pad, dh), _F32),
        mesh=mesh,
        scratch_types=[
            pltpu.VMEM_SHARED((n_pad + 8, dh), _F32),
            pltpu.VMEM((ib, ch), jnp.int32),
            pltpu.VMEM((ch, dh), _F32),
        ],
        compiler_params=pltpu.CompilerParams(use_tc_tiling_on_sc=False),
    )
    def k(m_h, idx_h, z_h, o_h, acc_s, idx_v, mbuf):
        c = lax.axis_index("c")
        s = lax.axis_index("s")
        m0 = c * ep
        o0 = c * n_pad

        pltpu.sync_copy(z_h, mbuf)

        def zacc(i, carry):
            pltpu.sync_copy(mbuf, acc_s.at[pl.ds(s * nrpt + i * ch, ch)])
            return carry

        lax.fori_loop(0, nz, zacc, 0)
        if zt:
            pltpu.sync_copy(mbuf.at[pl.ds(0, zt)],
                            acc_s.at[pl.ds(s * nrpt + nz * ch, zt)])
        plsc.subcore_barrier()

        def body(o, carry):
            pltpu.sync_copy(idx_h.at[pl.ds(s * rpt + o * ib, ib)], idx_v)
            for j in range(ib):
                g = o * ib + j
                pltpu.sync_copy(
                    m_h.at[pl.ds(m0 + (s * rpt + g) * ch, ch)], mbuf)
                pltpu.sync_copy(mbuf, acc_s.at[idx_v.at[j]], add=True)
            return carry

        lax.fori_loop(0, rpt // ib, body, 0)
        plsc.subcore_barrier()

        def ebody(i, carry):
            pltpu.sync_copy(acc_s.at[pl.ds(s * nrpt + i * ch, ch)], mbuf)
            pltpu.sync_copy(mbuf,
                            o_h.at[pl.ds(o0 + s * nrpt + i * ch, ch)])
            return carry

        lax.fori_loop(0, nz, ebody, 0)
        if zt:
            pltpu.sync_copy(acc_s.at[pl.ds(s * nrpt + nz * ch, zt)],
                            mbuf.at[pl.ds(0, zt)])
            pltpu.sync_copy(mbuf.at[pl.ds(0, zt)],
                            o_h.at[pl.ds(o0 + s * nrpt + nz * ch, zt)])

    return k(msg, idx, zeros)


_DEPTH = 6


def kernel(pose_feats, edge_index, edge_attr, node_timestamps, batch,
           edge_encoder, node_encoder, edge_classifier, edge_update,
           create_past_msgs, create_future_msgs, combine_future_past):
    n = pose_feats.shape[0]
    e = edge_index.shape[1]
    rows = edge_index[0].astype(jnp.int32)
    cols = edge_index[1].astype(jnp.int32)

    e_pad = -(-e // (_NS * _CH * 8)) * (_NS * _CH * 8)
    n_pad = -(-n // (_NS * 8)) * (_NS * 8)
    pad = jnp.full((e_pad - e,), n_pad, jnp.int32)
    cols_p = jnp.concatenate([cols, pad]).reshape(e_pad // _CH, _CH)
    rows_p = jnp.concatenate([rows, pad]).reshape(e_pad // _CH, _CH)

    node_l = _prep(node_encoder)
    edge_l = _prep(edge_encoder)
    cls_l = _prep(edge_classifier)
    eu_l = _prep(edge_update)
    fu_l = _prep(create_future_msgs)
    pa_l = _prep(create_past_msgs)
    cb_l = _prep(combine_future_past)

    nb = 2000
    ebk = 4000
    dx = node_l[-1][0].shape[1]

    tab0 = _mlp_tc(pose_feats, node_l, nb, table128=True)
    ea = _mlp_tc(edge_attr, edge_l, ebk)

    pack_i, pack_j = _sc_gather2(tab0, cols, rows)
    for depth in range(_DEPTH):
        if depth == _DEPTH - 1:
            ea = _edge_stage(pack_i, pack_j, ea, eu_l, fu_l, pa_l, dx,
                             ebk, e_pad, last=True)
            break
        upd, pa_stk, fu_stk = _edge_stage(
            pack_i, pack_j, ea, eu_l, fu_l, pa_l, dx, ebk, e_pad)
        dh = pa_stk.shape[2]
        mp = _sc_scatter(pa_stk.reshape(-1, dh), cols_p, n_pad)
        mf = _sc_scatter(fu_stk.reshape(-1, dh), rows_p, n_pad)
        ea = upd
        x_tab = _combine_tc(mp.reshape(2, n_pad, dh),
                            mf.reshape(2, n_pad, dh), tab0, cb_l, nb, n)
        pack_i, pack_j = _sc_gather2(x_tab, cols, rows)

    logits = _mlp_tc(ea, cls_l, ebk)
    x_enc = lax.slice(tab0, (0, 0), (n, dx))
    return (logits, x_enc)
